# Initial kernel scaffold; baseline (speedup 1.0000x reference)
import jax
import jax.numpy as jnp
from jax.experimental import pallas as pl


def kernel(s, w, neg, X):
    raise NotImplementedError("write your pallas kernel here")


# speedup vs baseline: 102.2429x; 102.2429x over previous
import functools

import jax
import jax.numpy as jnp
from jax import lax
from jax.experimental import pallas as pl
from jax.experimental.pallas import tpu as pltpu
from jax.experimental.pallas import tpu_sc as plsc

B = 16384
D = 128
WALK = 5
NC = 2
NS = 16
NW = NC * NS
NB = B // NW
C = 64
NCHUNK = NB // C


def _sc_body(s_hbm, w_hbm, x_hbm, lt_hbm, sidx, widx, xs_v, xw_v, lt_v, sem):
    wid = lax.axis_index("s") * NC + lax.axis_index("c")
    base = wid * NB
    pltpu.sync_copy(s_hbm.at[pl.ds(base, NB)], sidx)
    pltpu.sync_copy(w_hbm.at[pl.ds(base * WALK, NB * WALK)], widx)

    def chunk_body(c, carry):
        cp1 = pltpu.async_copy(x_hbm.at[sidx.at[pl.ds(c * C, C)]], xs_v, sem)
        cw = c * C * WALK
        cp2 = pltpu.async_copy(
            x_hbm.at[widx.at[pl.ds(cw, 128)]], xw_v.at[pl.ds(0, 128)], sem)
        cp3 = pltpu.async_copy(
            x_hbm.at[widx.at[pl.ds(cw + 128, 128)]], xw_v.at[pl.ds(128, 128)], sem)
        cp4 = pltpu.async_copy(
            x_hbm.at[widx.at[pl.ds(cw + 256, 64)]], xw_v.at[pl.ds(256, 64)], sem)
        cp1.wait()
        cp2.wait()
        cp3.wait()
        cp4.wait()

        lane_iota = lax.iota(jnp.int32, 16)

        def g_body(g, carry2):
            def l_body(i, accs):
                b = g * 16 + i
                xs = [xs_v[b, pl.ds(k * 16, 16)] for k in range(8)]
                new = []
                for j in range(WALK):
                    r = b * WALK + j
                    acc = xs[0] * xw_v[r, pl.ds(0, 16)]
                    for k in range(1, 8):
                        acc = acc + xs[k] * xw_v[r, pl.ds(k * 16, 16)]
                    lj = jnp.sum(acc)
                    new.append(jnp.where(lane_iota == i, lj, accs[j]))
                return tuple(new)

            accs = lax.fori_loop(
                0, 16, l_body,
                tuple(jnp.zeros((16,), jnp.float32) for _ in range(WALK)))
            for j in range(WALK):
                lt_v[j, pl.ds(c * C + g * 16, 16)] = accs[j]
            return carry2

        return lax.fori_loop(0, C // 16, g_body, carry)

    lax.fori_loop(0, NCHUNK, chunk_body, 0)
    pltpu.sync_copy(lt_v, lt_hbm.at[:, pl.ds(base, NB)])


_sc_logits = functools.partial(
    pl.kernel,
    mesh=plsc.VectorSubcoreMesh(core_axis_name="c", subcore_axis_name="s"),
    compiler_params=pltpu.CompilerParams(needs_layout_passes=False),
    out_type=jax.ShapeDtypeStruct((WALK, B), jnp.float32),
    scratch_types=[
        pltpu.VMEM((NB,), jnp.int32),
        pltpu.VMEM((NB * WALK,), jnp.int32),
        pltpu.VMEM((C, D), jnp.float32),
        pltpu.VMEM((C * WALK, D), jnp.float32),
        pltpu.VMEM((WALK, NB), jnp.float32),
        pltpu.SemaphoreType.DMA,
    ],
)(_sc_body)


def _tc_body(lt_ref, out_ref):
    lt = lt_ref[...]
    m = jnp.max(lt, axis=1, keepdims=True)
    ssum = jnp.sum(jnp.exp(lt - m), axis=1, keepdims=True)
    lse = m + jnp.log(ssum)
    k_const = jnp.sum(lse)
    out_ref[...] = (k_const - jnp.sum(lt, axis=0))[None, :]


def kernel(s, w, neg, X):
    del neg
    logits_t = _sc_logits(s, w.reshape(-1), X)
    out2 = pl.pallas_call(
        _tc_body,
        out_shape=jax.ShapeDtypeStruct((1, B), jnp.float32),
    )(logits_t)
    return out2.reshape(B)


# speedup vs baseline: 120.7905x; 1.1814x over previous
import functools

import jax
import jax.numpy as jnp
from jax import lax
from jax.experimental import pallas as pl
from jax.experimental.pallas import tpu as pltpu
from jax.experimental.pallas import tpu_sc as plsc

B = 16384
D = 128
WALK = 5
NC = 2
NS = 16
NW = NC * NS
NB = B // NW
C = 64
NCHUNK = NB // C


def _sc_body(s_hbm, w_hbm, x_hbm, lt_hbm, sidx, widx,
             xs0, xs1, xw0, xw1, lt_v, sem0, sem1):
    wid = lax.axis_index("s") * NC + lax.axis_index("c")
    base = wid * NB
    pltpu.sync_copy(s_hbm.at[pl.ds(base, NB)], sidx)
    pltpu.sync_copy(w_hbm.at[pl.ds(base * WALK, NB * WALK)], widx)

    xs_bufs = (xs0, xs1)
    xw_bufs = (xw0, xw1)
    sems = (sem0, sem1)

    def fire(c):
        xs_v, xw_v, sem = xs_bufs[c % 2], xw_bufs[c % 2], sems[c % 2]
        cw = c * C * WALK
        return (
            pltpu.async_copy(x_hbm.at[sidx.at[pl.ds(c * C, C)]], xs_v, sem),
            pltpu.async_copy(
                x_hbm.at[widx.at[pl.ds(cw, 128)]], xw_v.at[pl.ds(0, 128)], sem),
            pltpu.async_copy(
                x_hbm.at[widx.at[pl.ds(cw + 128, 128)]], xw_v.at[pl.ds(128, 128)],
                sem),
            pltpu.async_copy(
                x_hbm.at[widx.at[pl.ds(cw + 256, 64)]], xw_v.at[pl.ds(256, 64)],
                sem),
        )

    lane_iota = lax.iota(jnp.int32, 16)

    def compute(c):
        xs_v, xw_v = xs_bufs[c % 2], xw_bufs[c % 2]

        def g_body(g, carry2):
            def l_body(i, accs):
                b = g * 16 + i
                xs = [xs_v[b, pl.ds(k * 16, 16)] for k in range(8)]
                new = []
                for j in range(WALK):
                    r = b * WALK + j
                    acc = xs[0] * xw_v[r, pl.ds(0, 16)]
                    for k in range(1, 8):
                        acc = acc + xs[k] * xw_v[r, pl.ds(k * 16, 16)]
                    lj = jnp.sum(acc)
                    new.append(jnp.where(lane_iota == i, lj, accs[j]))
                return tuple(new)

            accs = lax.fori_loop(
                0, 16, l_body,
                tuple(jnp.zeros((16,), jnp.float32) for _ in range(WALK)))
            for j in range(WALK):
                lt_v[j, pl.ds(c * C + g * 16, 16)] = accs[j]
            return carry2

        lax.fori_loop(0, C // 16, g_body, 0)

    inflight = fire(0)
    for c in range(NCHUNK):
        for cp in inflight:
            cp.wait()
        if c + 1 < NCHUNK:
            inflight = fire(c + 1)
        compute(c)

    pltpu.sync_copy(lt_v, lt_hbm.at[:, pl.ds(base, NB)])


_sc_logits = functools.partial(
    pl.kernel,
    mesh=plsc.VectorSubcoreMesh(core_axis_name="c", subcore_axis_name="s"),
    compiler_params=pltpu.CompilerParams(needs_layout_passes=False),
    out_type=jax.ShapeDtypeStruct((WALK, B), jnp.float32),
    scratch_types=[
        pltpu.VMEM((NB,), jnp.int32),
        pltpu.VMEM((NB * WALK,), jnp.int32),
        pltpu.VMEM((C, D), jnp.float32),
        pltpu.VMEM((C, D), jnp.float32),
        pltpu.VMEM((C * WALK, D), jnp.float32),
        pltpu.VMEM((C * WALK, D), jnp.float32),
        pltpu.VMEM((WALK, NB), jnp.float32),
        pltpu.SemaphoreType.DMA,
        pltpu.SemaphoreType.DMA,
    ],
)(_sc_body)


def _tc_body(lt_ref, out_ref):
    lt = lt_ref[...]
    m = jnp.max(lt, axis=1, keepdims=True)
    ssum = jnp.sum(jnp.exp(lt - m), axis=1, keepdims=True)
    lse = m + jnp.log(ssum)
    k_const = jnp.sum(lse)
    out_ref[...] = (k_const - jnp.sum(lt, axis=0))[None, :]


def kernel(s, w, neg, X):
    del neg
    logits_t = _sc_logits(s, w.reshape(-1), X)
    out2 = pl.pallas_call(
        _tc_body,
        out_shape=jax.ShapeDtypeStruct((1, B), jnp.float32),
    )(logits_t)
    return out2.reshape(B)


# speedup vs baseline: 139.5016x; 1.1549x over previous
import functools

import jax
import jax.numpy as jnp
from jax import lax
from jax.experimental import pallas as pl
from jax.experimental.pallas import tpu as pltpu
from jax.experimental.pallas import tpu_sc as plsc

B = 16384
D = 128
WALK = 5
NC = 2
NS = 16
NW = NC * NS
NB = B // NW
C = 64
NCHUNK = NB // C


def _sc_body(s_hbm, w_hbm, x_hbm, t_hbm, st_hbm, sidx, widx,
             xs0, xs1, xw0, xw1, lt_v, t_v, st_v, sem0, sem1):
    wid = lax.axis_index("s") * NC + lax.axis_index("c")
    base = wid * NB
    pltpu.sync_copy(s_hbm.at[pl.ds(base, NB)], sidx)
    pltpu.sync_copy(w_hbm.at[:, pl.ds(base, NB)], widx)

    xs_bufs = (xs0, xs1)
    xw_bufs = (xw0, xw1)
    sems = (sem0, sem1)

    def fire(c):
        xs_v, xw_v, sem = xs_bufs[c % 2], xw_bufs[c % 2], sems[c % 2]
        cps = [pltpu.async_copy(x_hbm.at[sidx.at[pl.ds(c * C, C)]], xs_v, sem)]
        for j in range(WALK):
            cps.append(pltpu.async_copy(
                x_hbm.at[widx.at[j, pl.ds(c * C, C)]],
                xw_v.at[pl.ds(j * C, C)], sem))
        return cps

    lane_iota = lax.iota(jnp.int32, 16)

    def compute(c):
        xs_v, xw_v = xs_bufs[c % 2], xw_bufs[c % 2]

        def g_body(g, carry2):
            def l_body(i, accs):
                b = g * 16 + i
                xs = [xs_v[b, pl.ds(k * 16, 16)] for k in range(8)]
                new = []
                for j in range(WALK):
                    r = j * C + b
                    acc = xs[0] * xw_v[r, pl.ds(0, 16)]
                    for k in range(1, 8):
                        acc = acc + xs[k] * xw_v[r, pl.ds(k * 16, 16)]
                    lj = jnp.sum(acc)
                    new.append(jnp.where(lane_iota == i, lj, accs[j]))
                return tuple(new)

            accs = lax.fori_loop(
                0, 16, l_body,
                tuple(jnp.zeros((16,), jnp.float32) for _ in range(WALK)))
            for j in range(WALK):
                lt_v[j, pl.ds(c * C + g * 16, 16)] = accs[j]
            return carry2

        lax.fori_loop(0, C // 16, g_body, 0)

    inflight = fire(0)
    for c in range(NCHUNK):
        for cp in inflight:
            cp.wait()
        if c + 1 < NCHUNK:
            inflight = fire(c + 1)
        compute(c)

    stats = []
    for j in range(WALK):
        m = lt_v[j, pl.ds(0, 16)]
        for i in range(1, NB // 16):
            m = jnp.maximum(m, lt_v[j, pl.ds(i * 16, 16)])
        mj = jnp.max(m)
        se = jnp.zeros((16,), jnp.float32)
        for i in range(NB // 16):
            se = se + jnp.exp(lt_v[j, pl.ds(i * 16, 16)] - mj)
        stats.append((mj, jnp.sum(se)))

    def t_body(i, carry3):
        tv = lt_v[0, pl.ds(i * 16, 16)]
        for j in range(1, WALK):
            tv = tv + lt_v[j, pl.ds(i * 16, 16)]
        t_v[pl.ds(i * 16, 16)] = tv
        return carry3

    lax.fori_loop(0, NB // 16, t_body, 0)
    pltpu.sync_copy(t_v, t_hbm.at[0, pl.ds(base, NB)])

    st = jnp.zeros((16,), jnp.float32)
    for j in range(WALK):
        st = jnp.where(lane_iota == j, stats[j][0], st)
        st = jnp.where(lane_iota == (j + 8), stats[j][1], st)
    for q in range(8):
        st_v[0, pl.ds(q * 16, 16)] = st if q == 0 else jnp.zeros(
            (16,), jnp.float32)
    pltpu.sync_copy(st_v, st_hbm.at[wid])


_sc_partial = functools.partial(
    pl.kernel,
    mesh=plsc.VectorSubcoreMesh(core_axis_name="c", subcore_axis_name="s"),
    compiler_params=pltpu.CompilerParams(needs_layout_passes=False),
    out_type=(
        jax.ShapeDtypeStruct((1, B), jnp.float32),
        jax.ShapeDtypeStruct((NW, 1, 128), jnp.float32),
    ),
    scratch_types=[
        pltpu.VMEM((NB,), jnp.int32),
        pltpu.VMEM((WALK, NB), jnp.int32),
        pltpu.VMEM((C, D), jnp.float32),
        pltpu.VMEM((C, D), jnp.float32),
        pltpu.VMEM((C * WALK, D), jnp.float32),
        pltpu.VMEM((C * WALK, D), jnp.float32),
        pltpu.VMEM((WALK, NB), jnp.float32),
        pltpu.VMEM((NB,), jnp.float32),
        pltpu.VMEM((1, 128), jnp.float32),
        pltpu.SemaphoreType.DMA,
        pltpu.SemaphoreType.DMA,
    ],
)(_sc_body)


def _tc_body(t_ref, st_ref, out_ref):
    st = st_ref[...][:, 0, :]
    m = st[:, 0:WALK]
    se = st[:, 8:8 + WALK]
    gm = jnp.max(m, axis=0, keepdims=True)
    s_all = jnp.sum(se * jnp.exp(m - gm), axis=0, keepdims=True)
    k_const = jnp.sum(gm + jnp.log(s_all))
    out_ref[...] = k_const - t_ref[...]


def kernel(s, w, neg, X):
    del neg
    t, st = _sc_partial(s, w.T, X)
    out2 = pl.pallas_call(
        _tc_body,
        out_shape=jax.ShapeDtypeStruct((1, B), jnp.float32),
    )(t, st)
    return out2.reshape(B)
